# baseline (speedup 1.0000x reference)
import functools

import jax
import jax.numpy as jnp
from jax.experimental import pallas as pl
from jax.experimental.pallas import tpu as pltpu

_HBLK = 128


def _body(idx_ref, tab_ref, w1_ref, b1_ref, w2_ref, b2_ref, out_ref,
          prompt_ref, acc_ref):
    j = pl.program_id(0)

    @pl.when(j == 0)
    def _init():
        idx = idx_ref[:, :]
        cols = jax.lax.broadcasted_iota(
            jnp.int32, (idx.shape[0], tab_ref.shape[0]), 1)
        onehot = (idx == cols).astype(jnp.float32)
        prompt_ref[:, :] = jnp.dot(
            onehot, tab_ref[:, :], preferred_element_type=jnp.float32)
        acc_ref[:, :] = jnp.broadcast_to(b2_ref[:, :], acc_ref.shape)

    h = jnp.tanh(
        jnp.dot(prompt_ref[:, :], w1_ref[:, :],
                preferred_element_type=jnp.float32)
        + b1_ref[:, :]
    )
    acc_ref[:, :] += jnp.dot(h, w2_ref[:, :],
                             preferred_element_type=jnp.float32)

    @pl.when(j == pl.num_programs(0) - 1)
    def _finish():
        out_ref[:, :, :] = jnp.broadcast_to(acc_ref[:, :][None], out_ref.shape)


def kernel(tokens, batch_size, pre_prompt, embd_table, W1, b1, W2, b2):
    B = tokens.shape[0]
    P = pre_prompt.shape[0]
    D, H = W1.shape
    nblk = H // _HBLK
    grid = (nblk,)
    return pl.pallas_call(
        _body,
        grid=grid,
        in_specs=[
            pl.BlockSpec((P, 1), lambda j: (0, 0)),
            pl.BlockSpec((P, D), lambda j: (0, 0)),
            pl.BlockSpec((D, _HBLK), lambda j: (0, j)),
            pl.BlockSpec((1, _HBLK), lambda j: (0, j)),
            pl.BlockSpec((_HBLK, D), lambda j: (j, 0)),
            pl.BlockSpec((1, D), lambda j: (0, 0)),
        ],
        out_specs=pl.BlockSpec((B, P, D), lambda j: (0, 0, 0)),
        out_shape=jax.ShapeDtypeStruct((B, P, D), jnp.float32),
        scratch_shapes=[
            pltpu.VMEM((P, D), jnp.float32),
            pltpu.VMEM((P, D), jnp.float32),
        ],
    )(
        pre_prompt.reshape(P, 1),
        embd_table,
        W1,
        b1.reshape(1, H),
        W2,
        b2.reshape(1, D),
    )
